# Initial kernel scaffold; baseline (speedup 1.0000x reference)
import jax
import jax.numpy as jnp
from jax.experimental import pallas as pl


def kernel(x, adj_low, adj_high, adj_nd_low, adj_nd_high, Wl1, Wr1, b1, Wlh1, Wrh1, bh1, Wl2, Wr2, b2, Wlh2, Wrh2, bh2):
    raise NotImplementedError("write your pallas kernel here")


# speedup vs baseline: 5.7193x; 5.7193x over previous
import jax
import jax.numpy as jnp
from jax import lax
from jax.experimental import pallas as pl
from jax.experimental.pallas import tpu as pltpu
from jax.experimental.pallas import tpu_sc as plsc

N = 10000
E = 320000
D = 128
ND_LAMBDA = 0.5

NC = 2
NS = 16
NW = NC * NS
CH = 128
EPT = ((E // NW + CH - 1) // CH) * CH
CHUNKS = EPT // CH
E_PAD = EPT * NW
N_PAD = 10112
R = N_PAD // NS

_mesh = plsc.VectorSubcoreMesh(core_axis_name="c", subcore_axis_name="s")


def _deg_body(dstL_hbm, dstN_hbm, z128_hbm, ones_hbm, degL_out, degN_out,
              dst_v, ones_v, deg_sh):
    cid = lax.axis_index("c")
    sid = lax.axis_index("s")
    wid = sid * NC + cid
    pltpu.sync_copy(ones_hbm, ones_v)
    for dst_hbm, deg_out in ((dstL_hbm, degL_out), (dstN_hbm, degN_out)):
        pltpu.sync_copy(dst_hbm.at[wid], dst_v)
        pltpu.sync_copy(z128_hbm.at[pl.ds(sid * R, R)],
                        deg_sh.at[pl.ds(sid * R, R)])
        plsc.subcore_barrier()

        def step(j, carry):
            pltpu.sync_copy(ones_v, deg_sh.at[dst_v.at[j]], add=True)
            return carry

        lax.fori_loop(0, CHUNKS, step, 0)
        plsc.subcore_barrier()
        pltpu.sync_copy(deg_sh.at[pl.ds(sid * R, R)],
                        deg_out.at[cid, pl.ds(sid * R, R)])
        plsc.subcore_barrier()


_deg_sc = pl.kernel(
    _deg_body,
    out_type=(jax.ShapeDtypeStruct((NC, N_PAD, D), jnp.float32),
              jax.ShapeDtypeStruct((NC, N_PAD, D), jnp.float32)),
    mesh=_mesh,
    scratch_types=[
        pltpu.VMEM((CHUNKS, CH), jnp.int32),
        pltpu.VMEM((CH, D), jnp.float32),
        pltpu.VMEM_SHARED((N_PAD, D), jnp.float32),
    ],
)


def _agg_body(x_hbm, srcL_hbm, dstL_hbm, srcN_hbm, dstN_hbm, z128_hbm,
              aggL_out, aggN_out,
              src_v, dst_v, rowbuf, agg_sh, sem):
    cid = lax.axis_index("c")
    sid = lax.axis_index("s")
    wid = sid * NC + cid
    for src_hbm, dst_hbm, agg_out in ((srcL_hbm, dstL_hbm, aggL_out),
                                      (srcN_hbm, dstN_hbm, aggN_out)):
        pltpu.sync_copy(src_hbm.at[wid], src_v)
        pltpu.sync_copy(dst_hbm.at[wid], dst_v)
        pltpu.sync_copy(z128_hbm.at[pl.ds(sid * R, R)],
                        agg_sh.at[pl.ds(sid * R, R)])
        plsc.subcore_barrier()

        def step(j, carry):
            pltpu.async_copy(x_hbm.at[src_v.at[j]], rowbuf, sem).wait()
            pltpu.sync_copy(rowbuf, agg_sh.at[dst_v.at[j]], add=True)
            return carry

        lax.fori_loop(0, CHUNKS, step, 0)
        plsc.subcore_barrier()
        pltpu.sync_copy(agg_sh.at[pl.ds(sid * R, R)],
                        agg_out.at[cid, pl.ds(sid * R, R)])
        plsc.subcore_barrier()


_agg_sc = pl.kernel(
    _agg_body,
    out_type=(jax.ShapeDtypeStruct((NC, N_PAD, D), jnp.float32),
              jax.ShapeDtypeStruct((NC, N_PAD, D), jnp.float32)),
    mesh=_mesh,
    scratch_types=[
        pltpu.VMEM((CHUNKS, CH), jnp.int32),
        pltpu.VMEM((CHUNKS, CH), jnp.int32),
        pltpu.VMEM((CH, D), jnp.float32),
        pltpu.VMEM_SHARED((N_PAD, D), jnp.float32),
        pltpu.SemaphoreType.DMA,
    ],
)


def _combine_kernel(aggL_ref, aggN_ref, degL_ref, degN_ref, x_ref, w_ref,
                    b_ref, out_ref):
    aL = aggL_ref[0] + aggL_ref[1]
    aN = aggN_ref[0] + aggN_ref[1]
    dL = degL_ref[0, :, 0:1] + degL_ref[1, :, 0:1]
    dN = degN_ref[0, :, 0:1] + degN_ref[1, :, 0:1]
    mL = aL / jnp.maximum(dL, 1.0)
    mN = aN / jnp.maximum(dN, 1.0)
    acc = jnp.dot(mL, w_ref[0:D, :], preferred_element_type=jnp.float32)
    acc += jnp.dot(mN, w_ref[D:2 * D, :], preferred_element_type=jnp.float32)
    acc += jnp.dot(x_ref[...], w_ref[2 * D:3 * D, :],
                   preferred_element_type=jnp.float32)
    out_ref[...] = acc + b_ref[...]


_BLK = 1000


def _combine(aggL, aggN, degL, degN, x, w, b):
    grid = (N // _BLK,)
    return pl.pallas_call(
        _combine_kernel,
        grid=grid,
        in_specs=[
            pl.BlockSpec((NC, _BLK, D), lambda i: (0, i, 0)),
            pl.BlockSpec((NC, _BLK, D), lambda i: (0, i, 0)),
            pl.BlockSpec((NC, _BLK, D), lambda i: (0, i, 0)),
            pl.BlockSpec((NC, _BLK, D), lambda i: (0, i, 0)),
            pl.BlockSpec((_BLK, D), lambda i: (i, 0)),
            pl.BlockSpec((3 * D, D), lambda i: (0, 0)),
            pl.BlockSpec((1, D), lambda i: (0, 0)),
        ],
        out_specs=pl.BlockSpec((_BLK, D), lambda i: (i, 0)),
        out_shape=jax.ShapeDtypeStruct((N, D), jnp.float32),
    )(aggL, aggN, degL, degN, x, w, b)


def _prep_edges(edge_index):
    src = edge_index[0].astype(jnp.int32)
    dst = edge_index[1].astype(jnp.int32)
    pad = E_PAD - E
    ar = jnp.arange(pad, dtype=jnp.int32)
    pad_src = ar % N
    pad_dst = N + ar % (N_PAD - N)
    src_p = jnp.concatenate([src, pad_src]).reshape(NW, CHUNKS, CH)
    dst_p = jnp.concatenate([dst, pad_dst]).reshape(NW, CHUNKS, CH)
    return src_p, dst_p


def kernel(x, adj_low, adj_high, adj_nd_low, adj_nd_high,
           Wl1, Wr1, b1, Wlh1, Wrh1, bh1,
           Wl2, Wr2, b2, Wlh2, Wrh2, bh2):
    srcL, dstL = _prep_edges(adj_low)
    srcN, dstN = _prep_edges(adj_nd_low)
    z128 = jnp.zeros((N_PAD, D), jnp.float32)
    ones = jnp.ones((CH, D), jnp.float32)

    w1 = jnp.concatenate([Wl1, ND_LAMBDA * Wlh1, Wr1 + ND_LAMBDA * Wrh1], axis=0)
    c1 = (b1 + ND_LAMBDA * bh1).reshape(1, D)
    w2 = jnp.concatenate([Wl2, ND_LAMBDA * Wlh2, Wr2 + ND_LAMBDA * Wrh2], axis=0)
    c2 = (b2 + ND_LAMBDA * bh2).reshape(1, D)

    degL, degN = _deg_sc(dstL, dstN, z128, ones)

    wc = (jnp.stack([w1, w2]), jnp.stack([c1, c2]))

    def body(xcur, ws):
        w, c = ws
        aggL, aggN = _agg_sc(xcur, srcL, dstL, srcN, dstN, z128)
        hcur = _combine(aggL, aggN, degL, degN, xcur, w, c)
        return hcur, None

    out, _ = lax.scan(body, x, wc)
    return out


# speedup vs baseline: 7.0864x; 1.2390x over previous
import jax
import jax.numpy as jnp
from jax import lax
from jax.experimental import pallas as pl
from jax.experimental.pallas import tpu as pltpu
from jax.experimental.pallas import tpu_sc as plsc

N = 10000
E = 320000
D = 128
ND_LAMBDA = 0.5

NC = 2
NS = 16
NW = NC * NS
CH = 128
HCH = 64
EPT = ((E // NW + CH - 1) // CH) * CH
CHUNKS = EPT // CH
E_PAD = EPT * NW
N_PAD = 10112
R = N_PAD // NS

_mesh = plsc.VectorSubcoreMesh(core_axis_name="c", subcore_axis_name="s")


def _deg_body(dstL_hbm, dstN_hbm, z128_hbm, ones_hbm, degL_out, degN_out,
              dst_v, ones_v, deg_sh):
    cid = lax.axis_index("c")
    sid = lax.axis_index("s")
    wid = sid * NC + cid
    pltpu.sync_copy(ones_hbm, ones_v)
    for dst_hbm, deg_out in ((dstL_hbm, degL_out), (dstN_hbm, degN_out)):
        pltpu.sync_copy(dst_hbm.at[wid], dst_v)
        pltpu.sync_copy(z128_hbm.at[pl.ds(sid * R, R)],
                        deg_sh.at[pl.ds(sid * R, R)])
        plsc.subcore_barrier()

        def step(j, carry):
            pltpu.sync_copy(ones_v, deg_sh.at[dst_v.at[j]], add=True)
            return carry

        lax.fori_loop(0, CHUNKS, step, 0)
        plsc.subcore_barrier()
        pltpu.sync_copy(deg_sh.at[pl.ds(sid * R, R)],
                        deg_out.at[cid, pl.ds(sid * R, R)])
        plsc.subcore_barrier()


_deg_sc = pl.kernel(
    _deg_body,
    out_type=(jax.ShapeDtypeStruct((NC, N_PAD, D), jnp.float32),
              jax.ShapeDtypeStruct((NC, N_PAD, D), jnp.float32)),
    mesh=_mesh,
    scratch_types=[
        pltpu.VMEM((CHUNKS, CH), jnp.int32),
        pltpu.VMEM((CH, D), jnp.float32),
        pltpu.VMEM_SHARED((N_PAD, D), jnp.float32),
    ],
)


def _agg_body(x_hbm, srcL_hbm, dstL_hbm, srcN_hbm, dstN_hbm, z128_hbm,
              aggL_out, aggN_out,
              src_v, dst_v, buf_a, buf_b, agg_sh, sem_a, sem_b):
    cid = lax.axis_index("c")
    sid = lax.axis_index("s")
    wid = sid * NC + cid

    def start(j, h, buf, sem):
        pltpu.async_copy(x_hbm.at[src_v.at[j, pl.ds(h * HCH, HCH)]], buf, sem)

    def finish(j, h, buf, sem):
        pltpu.make_async_copy(x_hbm.at[src_v.at[j, pl.ds(h * HCH, HCH)]],
                              buf, sem).wait()
        pltpu.sync_copy(buf, agg_sh.at[dst_v.at[j, pl.ds(h * HCH, HCH)]],
                        add=True)

    for src_hbm, dst_hbm, agg_out in ((srcL_hbm, dstL_hbm, aggL_out),
                                      (srcN_hbm, dstN_hbm, aggN_out)):
        pltpu.sync_copy(src_hbm.at[wid], src_v)
        pltpu.sync_copy(dst_hbm.at[wid], dst_v)
        pltpu.sync_copy(z128_hbm.at[pl.ds(sid * R, R)],
                        agg_sh.at[pl.ds(sid * R, R)])
        plsc.subcore_barrier()

        start(0, 0, buf_a, sem_a)

        def step(j, carry):
            start(j, 1, buf_b, sem_b)
            finish(j, 0, buf_a, sem_a)
            start(j + 1, 0, buf_a, sem_a)
            finish(j, 1, buf_b, sem_b)
            return carry

        lax.fori_loop(0, CHUNKS - 1, step, 0)
        start(CHUNKS - 1, 1, buf_b, sem_b)
        finish(CHUNKS - 1, 0, buf_a, sem_a)
        finish(CHUNKS - 1, 1, buf_b, sem_b)
        plsc.subcore_barrier()
        pltpu.sync_copy(agg_sh.at[pl.ds(sid * R, R)],
                        agg_out.at[cid, pl.ds(sid * R, R)])
        plsc.subcore_barrier()


_agg_sc = pl.kernel(
    _agg_body,
    out_type=(jax.ShapeDtypeStruct((NC, N_PAD, D), jnp.float32),
              jax.ShapeDtypeStruct((NC, N_PAD, D), jnp.float32)),
    mesh=_mesh,
    scratch_types=[
        pltpu.VMEM((CHUNKS, CH), jnp.int32),
        pltpu.VMEM((CHUNKS, CH), jnp.int32),
        pltpu.VMEM((HCH, D), jnp.float32),
        pltpu.VMEM((HCH, D), jnp.float32),
        pltpu.VMEM_SHARED((N_PAD, D), jnp.float32),
        pltpu.SemaphoreType.DMA,
        pltpu.SemaphoreType.DMA,
    ],
)


def _combine_kernel(aggL_ref, aggN_ref, degL_ref, degN_ref, x_ref, w_ref,
                    b_ref, out_ref):
    aL = aggL_ref[0] + aggL_ref[1]
    aN = aggN_ref[0] + aggN_ref[1]
    dL = degL_ref[0, :, 0:1] + degL_ref[1, :, 0:1]
    dN = degN_ref[0, :, 0:1] + degN_ref[1, :, 0:1]
    mL = aL / jnp.maximum(dL, 1.0)
    mN = aN / jnp.maximum(dN, 1.0)
    acc = jnp.dot(mL, w_ref[0:D, :], preferred_element_type=jnp.float32)
    acc += jnp.dot(mN, w_ref[D:2 * D, :], preferred_element_type=jnp.float32)
    acc += jnp.dot(x_ref[...], w_ref[2 * D:3 * D, :],
                   preferred_element_type=jnp.float32)
    out_ref[...] = acc + b_ref[...]


_BLK = 1000


def _combine(aggL, aggN, degL, degN, x, w, b):
    grid = (N // _BLK,)
    return pl.pallas_call(
        _combine_kernel,
        grid=grid,
        in_specs=[
            pl.BlockSpec((NC, _BLK, D), lambda i: (0, i, 0)),
            pl.BlockSpec((NC, _BLK, D), lambda i: (0, i, 0)),
            pl.BlockSpec((NC, _BLK, D), lambda i: (0, i, 0)),
            pl.BlockSpec((NC, _BLK, D), lambda i: (0, i, 0)),
            pl.BlockSpec((_BLK, D), lambda i: (i, 0)),
            pl.BlockSpec((3 * D, D), lambda i: (0, 0)),
            pl.BlockSpec((1, D), lambda i: (0, 0)),
        ],
        out_specs=pl.BlockSpec((_BLK, D), lambda i: (i, 0)),
        out_shape=jax.ShapeDtypeStruct((N, D), jnp.float32),
    )(aggL, aggN, degL, degN, x, w, b)


def _prep_edges(edge_index):
    src = edge_index[0].astype(jnp.int32)
    dst = edge_index[1].astype(jnp.int32)
    pad = E_PAD - E
    ar = jnp.arange(pad, dtype=jnp.int32)
    pad_src = ar % N
    pad_dst = N + ar % (N_PAD - N)
    src_p = jnp.concatenate([src, pad_src]).reshape(NW, CHUNKS, CH)
    dst_p = jnp.concatenate([dst, pad_dst]).reshape(NW, CHUNKS, CH)
    return src_p, dst_p


def kernel(x, adj_low, adj_high, adj_nd_low, adj_nd_high,
           Wl1, Wr1, b1, Wlh1, Wrh1, bh1,
           Wl2, Wr2, b2, Wlh2, Wrh2, bh2):
    srcL, dstL = _prep_edges(adj_low)
    srcN, dstN = _prep_edges(adj_nd_low)
    z128 = jnp.zeros((N_PAD, D), jnp.float32)
    ones = jnp.ones((CH, D), jnp.float32)

    w1 = jnp.concatenate([Wl1, ND_LAMBDA * Wlh1, Wr1 + ND_LAMBDA * Wrh1], axis=0)
    c1 = (b1 + ND_LAMBDA * bh1).reshape(1, D)
    w2 = jnp.concatenate([Wl2, ND_LAMBDA * Wlh2, Wr2 + ND_LAMBDA * Wrh2], axis=0)
    c2 = (b2 + ND_LAMBDA * bh2).reshape(1, D)

    degL, degN = _deg_sc(dstL, dstN, z128, ones)

    wc = (jnp.stack([w1, w2]), jnp.stack([c1, c2]))

    def body(xcur, ws):
        w, c = ws
        aggL, aggN = _agg_sc(xcur, srcL, dstL, srcN, dstN, z128)
        hcur = _combine(aggL, aggN, degL, degN, xcur, w, c)
        return hcur, None

    out, _ = lax.scan(body, x, wc)
    return out


# speedup vs baseline: 7.1236x; 1.0052x over previous
import jax
import jax.numpy as jnp
from jax import lax
from jax.experimental import pallas as pl
from jax.experimental.pallas import tpu as pltpu
from jax.experimental.pallas import tpu_sc as plsc

N = 10000
E = 320000
D = 128
ND_LAMBDA = 0.5

NC = 2
NS = 16
NW = NC * NS
CH = 128
HCH = 64
EPT = ((E // NW + CH - 1) // CH) * CH
CHUNKS = EPT // CH
E_PAD = EPT * NW
N_PAD = 10112
R = N_PAD // NS

_mesh = plsc.VectorSubcoreMesh(core_axis_name="c", subcore_axis_name="s")


def _deg_body(dstL_hbm, dstN_hbm, z128_hbm, ones_hbm, degL_out, degN_out,
              dst_v, ones_v, deg_sh, sem_a, sem_b):
    cid = lax.axis_index("c")
    sid = lax.axis_index("s")
    wid = sid * NC + cid
    pltpu.sync_copy(ones_hbm, ones_v)

    def issue(j, h, sem):
        pltpu.async_copy(ones_v, deg_sh.at[dst_v.at[j, pl.ds(h * HCH, HCH)]],
                         sem, add=True)

    def wait(j, h, sem):
        pltpu.make_async_copy(
            ones_v, deg_sh.at[dst_v.at[j, pl.ds(h * HCH, HCH)]], sem).wait()

    for dst_hbm, deg_out in ((dstL_hbm, degL_out), (dstN_hbm, degN_out)):
        pltpu.sync_copy(dst_hbm.at[wid], dst_v)
        pltpu.sync_copy(z128_hbm.at[pl.ds(sid * R, R)],
                        deg_sh.at[pl.ds(sid * R, R)])
        plsc.subcore_barrier()

        issue(0, 0, sem_a)
        issue(0, 1, sem_b)

        def step(j, carry):
            wait(j, 0, sem_a)
            issue(j + 1, 0, sem_a)
            wait(j, 1, sem_b)
            issue(j + 1, 1, sem_b)
            return carry

        lax.fori_loop(0, CHUNKS - 1, step, 0)
        wait(CHUNKS - 1, 0, sem_a)
        wait(CHUNKS - 1, 1, sem_b)
        plsc.subcore_barrier()
        pltpu.sync_copy(deg_sh.at[pl.ds(sid * R, R)],
                        deg_out.at[cid, pl.ds(sid * R, R)])
        plsc.subcore_barrier()


_deg_sc = pl.kernel(
    _deg_body,
    out_type=(jax.ShapeDtypeStruct((NC, N_PAD, D), jnp.float32),
              jax.ShapeDtypeStruct((NC, N_PAD, D), jnp.float32)),
    mesh=_mesh,
    scratch_types=[
        pltpu.VMEM((CHUNKS, CH), jnp.int32),
        pltpu.VMEM((HCH, D), jnp.float32),
        pltpu.VMEM_SHARED((N_PAD, D), jnp.float32),
        pltpu.SemaphoreType.DMA,
        pltpu.SemaphoreType.DMA,
    ],
)


def _agg_body(x_hbm, srcL_hbm, dstL_hbm, srcN_hbm, dstN_hbm, z128_hbm,
              aggL_out, aggN_out,
              src_v, dst_v, buf_a, buf_b, agg_sh, sem_a, sem_b):
    cid = lax.axis_index("c")
    sid = lax.axis_index("s")
    wid = sid * NC + cid

    def start(j, h, buf, sem):
        pltpu.async_copy(x_hbm.at[src_v.at[j, pl.ds(h * HCH, HCH)]], buf, sem)

    def finish(j, h, buf, sem):
        pltpu.make_async_copy(x_hbm.at[src_v.at[j, pl.ds(h * HCH, HCH)]],
                              buf, sem).wait()
        pltpu.sync_copy(buf, agg_sh.at[dst_v.at[j, pl.ds(h * HCH, HCH)]],
                        add=True)

    for src_hbm, dst_hbm, agg_out in ((srcL_hbm, dstL_hbm, aggL_out),
                                      (srcN_hbm, dstN_hbm, aggN_out)):
        pltpu.sync_copy(src_hbm.at[wid], src_v)
        pltpu.sync_copy(dst_hbm.at[wid], dst_v)
        pltpu.sync_copy(z128_hbm.at[pl.ds(sid * R, R)],
                        agg_sh.at[pl.ds(sid * R, R)])
        plsc.subcore_barrier()

        start(0, 0, buf_a, sem_a)

        def step(j, carry):
            start(j, 1, buf_b, sem_b)
            finish(j, 0, buf_a, sem_a)
            start(j + 1, 0, buf_a, sem_a)
            finish(j, 1, buf_b, sem_b)
            return carry

        lax.fori_loop(0, CHUNKS - 1, step, 0)
        start(CHUNKS - 1, 1, buf_b, sem_b)
        finish(CHUNKS - 1, 0, buf_a, sem_a)
        finish(CHUNKS - 1, 1, buf_b, sem_b)
        plsc.subcore_barrier()
        pltpu.sync_copy(agg_sh.at[pl.ds(sid * R, R)],
                        agg_out.at[cid, pl.ds(sid * R, R)])
        plsc.subcore_barrier()


_agg_sc = pl.kernel(
    _agg_body,
    out_type=(jax.ShapeDtypeStruct((NC, N_PAD, D), jnp.float32),
              jax.ShapeDtypeStruct((NC, N_PAD, D), jnp.float32)),
    mesh=_mesh,
    scratch_types=[
        pltpu.VMEM((CHUNKS, CH), jnp.int32),
        pltpu.VMEM((CHUNKS, CH), jnp.int32),
        pltpu.VMEM((HCH, D), jnp.float32),
        pltpu.VMEM((HCH, D), jnp.float32),
        pltpu.VMEM_SHARED((N_PAD, D), jnp.float32),
        pltpu.SemaphoreType.DMA,
        pltpu.SemaphoreType.DMA,
    ],
)


def _combine_kernel(aggL_ref, aggN_ref, degL_ref, degN_ref, x_ref, w_ref,
                    b_ref, out_ref):
    aL = aggL_ref[0] + aggL_ref[1]
    aN = aggN_ref[0] + aggN_ref[1]
    dL = degL_ref[0, :, 0:1] + degL_ref[1, :, 0:1]
    dN = degN_ref[0, :, 0:1] + degN_ref[1, :, 0:1]
    mL = aL / jnp.maximum(dL, 1.0)
    mN = aN / jnp.maximum(dN, 1.0)
    acc = jnp.dot(mL, w_ref[0:D, :], preferred_element_type=jnp.float32)
    acc += jnp.dot(mN, w_ref[D:2 * D, :], preferred_element_type=jnp.float32)
    acc += jnp.dot(x_ref[...], w_ref[2 * D:3 * D, :],
                   preferred_element_type=jnp.float32)
    out_ref[...] = acc + b_ref[...]


_BLK = 1000


def _combine(aggL, aggN, degL, degN, x, w, b):
    grid = (N // _BLK,)
    return pl.pallas_call(
        _combine_kernel,
        grid=grid,
        in_specs=[
            pl.BlockSpec((NC, _BLK, D), lambda i: (0, i, 0)),
            pl.BlockSpec((NC, _BLK, D), lambda i: (0, i, 0)),
            pl.BlockSpec((NC, _BLK, D), lambda i: (0, i, 0)),
            pl.BlockSpec((NC, _BLK, D), lambda i: (0, i, 0)),
            pl.BlockSpec((_BLK, D), lambda i: (i, 0)),
            pl.BlockSpec((3 * D, D), lambda i: (0, 0)),
            pl.BlockSpec((1, D), lambda i: (0, 0)),
        ],
        out_specs=pl.BlockSpec((_BLK, D), lambda i: (i, 0)),
        out_shape=jax.ShapeDtypeStruct((N, D), jnp.float32),
    )(aggL, aggN, degL, degN, x, w, b)


def _prep_edges(edge_index):
    src = edge_index[0].astype(jnp.int32)
    dst = edge_index[1].astype(jnp.int32)
    pad = E_PAD - E
    ar = jnp.arange(pad, dtype=jnp.int32)
    pad_src = ar % N
    pad_dst = N + ar % (N_PAD - N)
    src_p = jnp.concatenate([src, pad_src]).reshape(NW, CHUNKS, CH)
    dst_p = jnp.concatenate([dst, pad_dst]).reshape(NW, CHUNKS, CH)
    return src_p, dst_p


def kernel(x, adj_low, adj_high, adj_nd_low, adj_nd_high,
           Wl1, Wr1, b1, Wlh1, Wrh1, bh1,
           Wl2, Wr2, b2, Wlh2, Wrh2, bh2):
    srcL, dstL = _prep_edges(adj_low)
    srcN, dstN = _prep_edges(adj_nd_low)
    z128 = jnp.zeros((N_PAD, D), jnp.float32)
    ones = jnp.ones((HCH, D), jnp.float32)

    w1 = jnp.concatenate([Wl1, ND_LAMBDA * Wlh1, Wr1 + ND_LAMBDA * Wrh1], axis=0)
    c1 = (b1 + ND_LAMBDA * bh1).reshape(1, D)
    w2 = jnp.concatenate([Wl2, ND_LAMBDA * Wlh2, Wr2 + ND_LAMBDA * Wrh2], axis=0)
    c2 = (b2 + ND_LAMBDA * bh2).reshape(1, D)

    degL, degN = _deg_sc(dstL, dstN, z128, ones)

    wc = (jnp.stack([w1, w2]), jnp.stack([c1, c2]))

    def body(xcur, ws):
        w, c = ws
        aggL, aggN = _agg_sc(xcur, srcL, dstL, srcN, dstN, z128)
        hcur = _combine(aggL, aggN, degL, degN, xcur, w, c)
        return hcur, None

    out, _ = lax.scan(body, x, wc)
    return out


# speedup vs baseline: 7.3124x; 1.0265x over previous
import jax
import jax.numpy as jnp
from jax import lax
from jax.experimental import pallas as pl
from jax.experimental.pallas import tpu as pltpu
from jax.experimental.pallas import tpu_sc as plsc

N = 10000
E = 320000
D = 128
ND_LAMBDA = 0.5

NC = 2
NS = 16
NW = NC * NS
CH = 128
HCH = 64
EPT = ((E // NW + CH - 1) // CH) * CH
CHUNKS = EPT // CH
E_PAD = EPT * NW
N_PAD = 10112
R = N_PAD // NS

_mesh = plsc.VectorSubcoreMesh(core_axis_name="c", subcore_axis_name="s")


def _zero_acc(z128_hbm, acc_sh, sid):
    pltpu.sync_copy(z128_hbm.at[pl.ds(sid * R, R)],
                    acc_sh.at[pl.ds(sid * R, R)])


def _writeback(acc_sh, out, cid, sid):
    pltpu.sync_copy(acc_sh.at[pl.ds(sid * R, R)],
                    out.at[cid, pl.ds(sid * R, R)])


def _deg_pass(dst_v, ones_v, deg_sh, sem_a, sem_b):
    def issue(j, h, sem):
        pltpu.async_copy(ones_v, deg_sh.at[dst_v.at[j, pl.ds(h * HCH, HCH)]],
                         sem, add=True)

    def wait(j, h, sem):
        pltpu.make_async_copy(
            ones_v, deg_sh.at[dst_v.at[j, pl.ds(h * HCH, HCH)]], sem).wait()

    issue(0, 0, sem_a)
    issue(0, 1, sem_b)

    def step(j, carry):
        wait(j, 0, sem_a)
        issue(j + 1, 0, sem_a)
        wait(j, 1, sem_b)
        issue(j + 1, 1, sem_b)
        return carry

    lax.fori_loop(0, CHUNKS - 1, step, 0)
    wait(CHUNKS - 1, 0, sem_a)
    wait(CHUNKS - 1, 1, sem_b)


def _agg_pass(x_hbm, src_v, dst_v, buf_a, buf_b, agg_sh, sem_a, sem_b):
    def start(j, h, buf, sem):
        pltpu.async_copy(x_hbm.at[src_v.at[j, pl.ds(h * HCH, HCH)]], buf, sem)

    def finish(j, h, buf, sem):
        pltpu.make_async_copy(x_hbm.at[src_v.at[j, pl.ds(h * HCH, HCH)]],
                              buf, sem).wait()
        pltpu.sync_copy(buf, agg_sh.at[dst_v.at[j, pl.ds(h * HCH, HCH)]],
                        add=True)

    start(0, 0, buf_a, sem_a)

    def step(j, carry):
        start(j, 1, buf_b, sem_b)
        finish(j, 0, buf_a, sem_a)
        start(j + 1, 0, buf_a, sem_a)
        finish(j, 1, buf_b, sem_b)
        return carry

    lax.fori_loop(0, CHUNKS - 1, step, 0)
    start(CHUNKS - 1, 1, buf_b, sem_b)
    finish(CHUNKS - 1, 0, buf_a, sem_a)
    finish(CHUNKS - 1, 1, buf_b, sem_b)


def _l1_body(x_hbm, srcL_hbm, dstL_hbm, srcN_hbm, dstN_hbm, z128_hbm,
             ones_hbm, degL_out, degN_out, aggL_out, aggN_out,
             src_v, dst_v, buf_a, buf_b, acc_sh, sem_a, sem_b):
    cid = lax.axis_index("c")
    sid = lax.axis_index("s")
    wid = sid * NC + cid
    pltpu.sync_copy(ones_hbm, buf_a)

    for dst_hbm, deg_out in ((dstL_hbm, degL_out), (dstN_hbm, degN_out)):
        pltpu.sync_copy(dst_hbm.at[wid], dst_v)
        _zero_acc(z128_hbm, acc_sh, sid)
        plsc.subcore_barrier()
        _deg_pass(dst_v, buf_a, acc_sh, sem_a, sem_b)
        plsc.subcore_barrier()
        _writeback(acc_sh, deg_out, cid, sid)
        plsc.subcore_barrier()

    for src_hbm, dst_hbm, agg_out in ((srcL_hbm, dstL_hbm, aggL_out),
                                      (srcN_hbm, dstN_hbm, aggN_out)):
        pltpu.sync_copy(src_hbm.at[wid], src_v)
        pltpu.sync_copy(dst_hbm.at[wid], dst_v)
        _zero_acc(z128_hbm, acc_sh, sid)
        plsc.subcore_barrier()
        _agg_pass(x_hbm, src_v, dst_v, buf_a, buf_b, acc_sh, sem_a, sem_b)
        plsc.subcore_barrier()
        _writeback(acc_sh, agg_out, cid, sid)
        plsc.subcore_barrier()


_l1_sc = pl.kernel(
    _l1_body,
    out_type=(jax.ShapeDtypeStruct((NC, N_PAD, D), jnp.float32),
              jax.ShapeDtypeStruct((NC, N_PAD, D), jnp.float32),
              jax.ShapeDtypeStruct((NC, N_PAD, D), jnp.float32),
              jax.ShapeDtypeStruct((NC, N_PAD, D), jnp.float32)),
    mesh=_mesh,
    scratch_types=[
        pltpu.VMEM((CHUNKS, CH), jnp.int32),
        pltpu.VMEM((CHUNKS, CH), jnp.int32),
        pltpu.VMEM((HCH, D), jnp.float32),
        pltpu.VMEM((HCH, D), jnp.float32),
        pltpu.VMEM_SHARED((N_PAD, D), jnp.float32),
        pltpu.SemaphoreType.DMA,
        pltpu.SemaphoreType.DMA,
    ],
)


def _agg_body(x_hbm, srcL_hbm, dstL_hbm, srcN_hbm, dstN_hbm, z128_hbm,
              aggL_out, aggN_out,
              src_v, dst_v, buf_a, buf_b, agg_sh, sem_a, sem_b):
    cid = lax.axis_index("c")
    sid = lax.axis_index("s")
    wid = sid * NC + cid

    for src_hbm, dst_hbm, agg_out in ((srcL_hbm, dstL_hbm, aggL_out),
                                      (srcN_hbm, dstN_hbm, aggN_out)):
        pltpu.sync_copy(src_hbm.at[wid], src_v)
        pltpu.sync_copy(dst_hbm.at[wid], dst_v)
        _zero_acc(z128_hbm, agg_sh, sid)
        plsc.subcore_barrier()
        _agg_pass(x_hbm, src_v, dst_v, buf_a, buf_b, agg_sh, sem_a, sem_b)
        plsc.subcore_barrier()
        _writeback(agg_sh, agg_out, cid, sid)
        plsc.subcore_barrier()


_agg_sc = pl.kernel(
    _agg_body,
    out_type=(jax.ShapeDtypeStruct((NC, N_PAD, D), jnp.float32),
              jax.ShapeDtypeStruct((NC, N_PAD, D), jnp.float32)),
    mesh=_mesh,
    scratch_types=[
        pltpu.VMEM((CHUNKS, CH), jnp.int32),
        pltpu.VMEM((CHUNKS, CH), jnp.int32),
        pltpu.VMEM((HCH, D), jnp.float32),
        pltpu.VMEM((HCH, D), jnp.float32),
        pltpu.VMEM_SHARED((N_PAD, D), jnp.float32),
        pltpu.SemaphoreType.DMA,
        pltpu.SemaphoreType.DMA,
    ],
)


def _combine_kernel(aggL_ref, aggN_ref, degL_ref, degN_ref, x_ref, w_ref,
                    b_ref, out_ref):
    aL = aggL_ref[0] + aggL_ref[1]
    aN = aggN_ref[0] + aggN_ref[1]
    dL = degL_ref[0, :, 0:1] + degL_ref[1, :, 0:1]
    dN = degN_ref[0, :, 0:1] + degN_ref[1, :, 0:1]
    mL = aL / jnp.maximum(dL, 1.0)
    mN = aN / jnp.maximum(dN, 1.0)
    acc = jnp.dot(mL, w_ref[0:D, :], preferred_element_type=jnp.float32)
    acc += jnp.dot(mN, w_ref[D:2 * D, :], preferred_element_type=jnp.float32)
    acc += jnp.dot(x_ref[...], w_ref[2 * D:3 * D, :],
                   preferred_element_type=jnp.float32)
    out_ref[...] = acc + b_ref[...]


_BLK = 1000


def _combine(aggL, aggN, degL, degN, x, w, b):
    grid = (N // _BLK,)
    return pl.pallas_call(
        _combine_kernel,
        grid=grid,
        in_specs=[
            pl.BlockSpec((NC, _BLK, D), lambda i: (0, i, 0)),
            pl.BlockSpec((NC, _BLK, D), lambda i: (0, i, 0)),
            pl.BlockSpec((NC, _BLK, D), lambda i: (0, i, 0)),
            pl.BlockSpec((NC, _BLK, D), lambda i: (0, i, 0)),
            pl.BlockSpec((_BLK, D), lambda i: (i, 0)),
            pl.BlockSpec((3 * D, D), lambda i: (0, 0)),
            pl.BlockSpec((1, D), lambda i: (0, 0)),
        ],
        out_specs=pl.BlockSpec((_BLK, D), lambda i: (i, 0)),
        out_shape=jax.ShapeDtypeStruct((N, D), jnp.float32),
    )(aggL, aggN, degL, degN, x, w, b)


def _prep_edges(edge_index):
    src = edge_index[0].astype(jnp.int32)
    dst = edge_index[1].astype(jnp.int32)
    pad = E_PAD - E
    ar = jnp.arange(pad, dtype=jnp.int32)
    pad_src = ar % N
    pad_dst = N + ar % (N_PAD - N)
    src_p = jnp.concatenate([src, pad_src]).reshape(NW, CHUNKS, CH)
    dst_p = jnp.concatenate([dst, pad_dst]).reshape(NW, CHUNKS, CH)
    return src_p, dst_p


def kernel(x, adj_low, adj_high, adj_nd_low, adj_nd_high,
           Wl1, Wr1, b1, Wlh1, Wrh1, bh1,
           Wl2, Wr2, b2, Wlh2, Wrh2, bh2):
    srcL, dstL = _prep_edges(adj_low)
    srcN, dstN = _prep_edges(adj_nd_low)
    z128 = jnp.zeros((N_PAD, D), jnp.float32)
    ones = jnp.ones((HCH, D), jnp.float32)

    w1 = jnp.concatenate([Wl1, ND_LAMBDA * Wlh1, Wr1 + ND_LAMBDA * Wrh1], axis=0)
    c1 = (b1 + ND_LAMBDA * bh1).reshape(1, D)
    w2 = jnp.concatenate([Wl2, ND_LAMBDA * Wlh2, Wr2 + ND_LAMBDA * Wrh2], axis=0)
    c2 = (b2 + ND_LAMBDA * bh2).reshape(1, D)

    degL, degN, aggL1, aggN1 = _l1_sc(x, srcL, dstL, srcN, dstN, z128, ones)
    h1 = _combine(aggL1, aggN1, degL, degN, x, w1, c1)
    aggL2, aggN2 = _agg_sc(h1, srcL, dstL, srcN, dstN, z128)
    return _combine(aggL2, aggN2, degL, degN, h1, w2, c2)


# speedup vs baseline: 7.4725x; 1.0219x over previous
import jax
import jax.numpy as jnp
from jax import lax
from jax.experimental import pallas as pl
from jax.experimental.pallas import tpu as pltpu
from jax.experimental.pallas import tpu_sc as plsc

N = 10000
E = 320000
D = 128
ND_LAMBDA = 0.5

NC = 2
NS = 16
NW = NC * NS
CH = 128
HCH = 64
EPT = ((E // NW + CH - 1) // CH) * CH
CHUNKS = EPT // CH
E_PAD = EPT * NW
N_PAD = 10112
R = N_PAD // NS

_mesh = plsc.VectorSubcoreMesh(core_axis_name="c", subcore_axis_name="s")


def _zero_acc(z128_hbm, acc_sh, sid):
    pltpu.sync_copy(z128_hbm.at[pl.ds(sid * R, R)],
                    acc_sh.at[pl.ds(sid * R, R)])


def _writeback(acc_sh, out, cid, sid):
    pltpu.sync_copy(acc_sh.at[pl.ds(sid * R, R)],
                    out.at[cid, pl.ds(sid * R, R)])


def _deg_pass(dst_v, ones_v, deg_sh, sem_a, sem_b):
    def issue(j, h, sem):
        pltpu.async_copy(ones_v, deg_sh.at[dst_v.at[j, pl.ds(h * HCH, HCH)]],
                         sem, add=True)

    def wait(j, h, sem):
        pltpu.make_async_copy(
            ones_v, deg_sh.at[dst_v.at[j, pl.ds(h * HCH, HCH)]], sem).wait()

    issue(0, 0, sem_a)
    issue(0, 1, sem_b)

    def step(j, carry):
        wait(j, 0, sem_a)
        issue(j + 1, 0, sem_a)
        wait(j, 1, sem_b)
        issue(j + 1, 1, sem_b)
        return carry

    lax.fori_loop(0, CHUNKS - 1, step, 0)
    wait(CHUNKS - 1, 0, sem_a)
    wait(CHUNKS - 1, 1, sem_b)


def _agg_pass(x_hbm, src_v, dst_v, buf_a, buf_b, agg_sh, sem_a, sem_b):
    def start(j, h, buf, sem):
        pltpu.async_copy(x_hbm.at[src_v.at[j, pl.ds(h * HCH, HCH)]], buf, sem)

    def finish(j, h, buf, sem):
        pltpu.make_async_copy(x_hbm.at[src_v.at[j, pl.ds(h * HCH, HCH)]],
                              buf, sem).wait()
        pltpu.sync_copy(buf, agg_sh.at[dst_v.at[j, pl.ds(h * HCH, HCH)]],
                        add=True)

    start(0, 0, buf_a, sem_a)

    def step(j, carry):
        start(j, 1, buf_b, sem_b)
        finish(j, 0, buf_a, sem_a)
        start(j + 1, 0, buf_a, sem_a)
        finish(j, 1, buf_b, sem_b)
        return carry

    lax.fori_loop(0, CHUNKS - 1, step, 0)
    start(CHUNKS - 1, 1, buf_b, sem_b)
    finish(CHUNKS - 1, 0, buf_a, sem_a)
    finish(CHUNKS - 1, 1, buf_b, sem_b)


def _l1_body(x_hbm, srcL_hbm, dstL_hbm, srcN_hbm, dstN_hbm, z128_hbm,
             ones2_hbm, deg_out, aggL_out, aggN_out,
             src_v, dst_v, buf_a, buf_b, acc_sh, sem_a, sem_b):
    cid = lax.axis_index("c")
    sid = lax.axis_index("s")
    wid = sid * NC + cid
    pltpu.sync_copy(ones2_hbm.at[0], buf_a)
    pltpu.sync_copy(ones2_hbm.at[1], buf_b)

    _zero_acc(z128_hbm, acc_sh, sid)
    pltpu.sync_copy(dstL_hbm.at[wid], dst_v)
    plsc.subcore_barrier()
    _deg_pass(dst_v, buf_a, acc_sh, sem_a, sem_b)
    pltpu.sync_copy(dstN_hbm.at[wid], dst_v)
    _deg_pass(dst_v, buf_b, acc_sh, sem_a, sem_b)
    plsc.subcore_barrier()
    _writeback(acc_sh, deg_out, cid, sid)
    plsc.subcore_barrier()

    for src_hbm, dst_hbm, agg_out in ((srcL_hbm, dstL_hbm, aggL_out),
                                      (srcN_hbm, dstN_hbm, aggN_out)):
        pltpu.sync_copy(src_hbm.at[wid], src_v)
        pltpu.sync_copy(dst_hbm.at[wid], dst_v)
        _zero_acc(z128_hbm, acc_sh, sid)
        plsc.subcore_barrier()
        _agg_pass(x_hbm, src_v, dst_v, buf_a, buf_b, acc_sh, sem_a, sem_b)
        plsc.subcore_barrier()
        _writeback(acc_sh, agg_out, cid, sid)
        plsc.subcore_barrier()


_l1_sc = pl.kernel(
    _l1_body,
    out_type=(jax.ShapeDtypeStruct((NC, N_PAD, D), jnp.float32),
              jax.ShapeDtypeStruct((NC, N_PAD, D), jnp.float32),
              jax.ShapeDtypeStruct((NC, N_PAD, D), jnp.float32)),
    mesh=_mesh,
    scratch_types=[
        pltpu.VMEM((CHUNKS, CH), jnp.int32),
        pltpu.VMEM((CHUNKS, CH), jnp.int32),
        pltpu.VMEM((HCH, D), jnp.float32),
        pltpu.VMEM((HCH, D), jnp.float32),
        pltpu.VMEM_SHARED((N_PAD, D), jnp.float32),
        pltpu.SemaphoreType.DMA,
        pltpu.SemaphoreType.DMA,
    ],
)


def _agg_body(x_hbm, srcL_hbm, dstL_hbm, srcN_hbm, dstN_hbm, z128_hbm,
              aggL_out, aggN_out,
              src_v, dst_v, buf_a, buf_b, agg_sh, sem_a, sem_b):
    cid = lax.axis_index("c")
    sid = lax.axis_index("s")
    wid = sid * NC + cid

    for src_hbm, dst_hbm, agg_out in ((srcL_hbm, dstL_hbm, aggL_out),
                                      (srcN_hbm, dstN_hbm, aggN_out)):
        pltpu.sync_copy(src_hbm.at[wid], src_v)
        pltpu.sync_copy(dst_hbm.at[wid], dst_v)
        _zero_acc(z128_hbm, agg_sh, sid)
        plsc.subcore_barrier()
        _agg_pass(x_hbm, src_v, dst_v, buf_a, buf_b, agg_sh, sem_a, sem_b)
        plsc.subcore_barrier()
        _writeback(agg_sh, agg_out, cid, sid)
        plsc.subcore_barrier()


_agg_sc = pl.kernel(
    _agg_body,
    out_type=(jax.ShapeDtypeStruct((NC, N_PAD, D), jnp.float32),
              jax.ShapeDtypeStruct((NC, N_PAD, D), jnp.float32)),
    mesh=_mesh,
    scratch_types=[
        pltpu.VMEM((CHUNKS, CH), jnp.int32),
        pltpu.VMEM((CHUNKS, CH), jnp.int32),
        pltpu.VMEM((HCH, D), jnp.float32),
        pltpu.VMEM((HCH, D), jnp.float32),
        pltpu.VMEM_SHARED((N_PAD, D), jnp.float32),
        pltpu.SemaphoreType.DMA,
        pltpu.SemaphoreType.DMA,
    ],
)


def _combine_kernel(aggL_ref, aggN_ref, deg_ref, x_ref, w_ref,
                    b_ref, out_ref):
    aL = aggL_ref[0] + aggL_ref[1]
    aN = aggN_ref[0] + aggN_ref[1]
    dL = deg_ref[0, :, 0:1] + deg_ref[1, :, 0:1]
    dN = deg_ref[0, :, 64:65] + deg_ref[1, :, 64:65]
    mL = aL / jnp.maximum(dL, 1.0)
    mN = aN / jnp.maximum(dN, 1.0)
    acc = jnp.dot(mL, w_ref[0:D, :], preferred_element_type=jnp.float32)
    acc += jnp.dot(mN, w_ref[D:2 * D, :], preferred_element_type=jnp.float32)
    acc += jnp.dot(x_ref[...], w_ref[2 * D:3 * D, :],
                   preferred_element_type=jnp.float32)
    out_ref[...] = acc + b_ref[...]


_BLK = 1000


def _combine(aggL, aggN, deg, x, w, b):
    grid = (N // _BLK,)
    return pl.pallas_call(
        _combine_kernel,
        grid=grid,
        in_specs=[
            pl.BlockSpec((NC, _BLK, D), lambda i: (0, i, 0)),
            pl.BlockSpec((NC, _BLK, D), lambda i: (0, i, 0)),
            pl.BlockSpec((NC, _BLK, D), lambda i: (0, i, 0)),
            pl.BlockSpec((_BLK, D), lambda i: (i, 0)),
            pl.BlockSpec((3 * D, D), lambda i: (0, 0)),
            pl.BlockSpec((1, D), lambda i: (0, 0)),
        ],
        out_specs=pl.BlockSpec((_BLK, D), lambda i: (i, 0)),
        out_shape=jax.ShapeDtypeStruct((N, D), jnp.float32),
    )(aggL, aggN, deg, x, w, b)


def _prep_edges(edge_index):
    src = edge_index[0].astype(jnp.int32)
    dst = edge_index[1].astype(jnp.int32)
    pad = E_PAD - E
    ar = jnp.arange(pad, dtype=jnp.int32)
    pad_src = ar % N
    pad_dst = N + ar % (N_PAD - N)
    src_p = jnp.concatenate([src, pad_src]).reshape(NW, CHUNKS, CH)
    dst_p = jnp.concatenate([dst, pad_dst]).reshape(NW, CHUNKS, CH)
    return src_p, dst_p


def kernel(x, adj_low, adj_high, adj_nd_low, adj_nd_high,
           Wl1, Wr1, b1, Wlh1, Wrh1, bh1,
           Wl2, Wr2, b2, Wlh2, Wrh2, bh2):
    srcL, dstL = _prep_edges(adj_low)
    srcN, dstN = _prep_edges(adj_nd_low)
    z128 = jnp.zeros((N_PAD, D), jnp.float32)
    lane = jnp.arange(D, dtype=jnp.int32)
    ones2 = jnp.stack([jnp.where(lane < 64, 1.0, 0.0),
                       jnp.where(lane < 64, 0.0, 1.0)]).astype(jnp.float32)
    ones2 = jnp.broadcast_to(ones2[:, None, :], (2, HCH, D))

    w1 = jnp.concatenate([Wl1, ND_LAMBDA * Wlh1, Wr1 + ND_LAMBDA * Wrh1], axis=0)
    c1 = (b1 + ND_LAMBDA * bh1).reshape(1, D)
    w2 = jnp.concatenate([Wl2, ND_LAMBDA * Wlh2, Wr2 + ND_LAMBDA * Wrh2], axis=0)
    c2 = (b2 + ND_LAMBDA * bh2).reshape(1, D)

    deg, aggL1, aggN1 = _l1_sc(x, srcL, dstL, srcN, dstN, z128, ones2)
    h1 = _combine(aggL1, aggN1, deg, x, w1, c1)
    aggL2, aggN2 = _agg_sc(h1, srcL, dstL, srcN, dstN, z128)
    return _combine(aggL2, aggN2, deg, h1, w2, c2)


# speedup vs baseline: 8.4496x; 1.1308x over previous
import jax
import jax.numpy as jnp
from jax import lax
from jax.experimental import pallas as pl
from jax.experimental.pallas import tpu as pltpu
from jax.experimental.pallas import tpu_sc as plsc

N = 10000
E = 320000
D = 128
ND_LAMBDA = 0.5

NC = 2
NS = 16
NW = NC * NS
CH = 128
HCH = 64
EPT = ((E // NW + CH - 1) // CH) * CH
CHUNKS = EPT // CH
E_PAD = EPT * NW
N_PAD = 10112
R = N_PAD // NS

_mesh = plsc.VectorSubcoreMesh(core_axis_name="c", subcore_axis_name="s")


def _zero_acc(z128_hbm, acc_sh, sid):
    pltpu.sync_copy(z128_hbm.at[pl.ds(sid * R, R)],
                    acc_sh.at[pl.ds(sid * R, R)])


def _writeback(acc_sh, out, cid, sid):
    pltpu.sync_copy(acc_sh.at[pl.ds(sid * R, R)],
                    out.at[cid, pl.ds(sid * R, R)])


QCH = 32
NQ = CH // QCH


def _deg_pass(dst_v, ones_v, deg_sh, sems):
    def issue(j, q, sem):
        pltpu.async_copy(ones_v, deg_sh.at[dst_v.at[j, pl.ds(q * QCH, QCH)]],
                         sem, add=True)

    def wait(j, q, sem):
        pltpu.make_async_copy(
            ones_v, deg_sh.at[dst_v.at[j, pl.ds(q * QCH, QCH)]], sem).wait()

    for q in range(NQ):
        issue(0, q, sems[q])

    def step(j, carry):
        for q in range(NQ):
            wait(j, q, sems[q])
            issue(j + 1, q, sems[q])
        return carry

    lax.fori_loop(0, CHUNKS - 1, step, 0)
    for q in range(NQ):
        wait(CHUNKS - 1, q, sems[q])


def _agg_pass(x_hbm, src_v, dst_v, bufs, agg_sh, sems):
    def start(j, q, sem):
        pltpu.async_copy(x_hbm.at[src_v.at[j, pl.ds(q * QCH, QCH)]],
                         bufs[q], sem)

    def finish(j, q, sem):
        pltpu.make_async_copy(x_hbm.at[src_v.at[j, pl.ds(q * QCH, QCH)]],
                              bufs[q], sem).wait()
        pltpu.sync_copy(bufs[q],
                        agg_sh.at[dst_v.at[j, pl.ds(q * QCH, QCH)]],
                        add=True)

    for q in range(NQ):
        start(0, q, sems[q])

    def step(j, carry):
        for q in range(NQ):
            finish(j, q, sems[q])
            start(j + 1, q, sems[q])
        return carry

    lax.fori_loop(0, CHUNKS - 1, step, 0)
    for q in range(NQ):
        finish(CHUNKS - 1, q, sems[q])


def _l1_body(x_hbm, srcL_hbm, dstL_hbm, srcN_hbm, dstN_hbm, z128_hbm,
             ones2_hbm, deg_out, aggL_out, aggN_out,
             src_v, dst_v, buf_a, buf_b, buf_c, buf_d, acc_sh,
             sem_a, sem_b, sem_c, sem_d):
    cid = lax.axis_index("c")
    sid = lax.axis_index("s")
    wid = sid * NC + cid
    pltpu.sync_copy(ones2_hbm.at[0], buf_a)
    pltpu.sync_copy(ones2_hbm.at[1], buf_b)

    sems = (sem_a, sem_b, sem_c, sem_d)
    bufs = (buf_a, buf_b, buf_c, buf_d)
    _zero_acc(z128_hbm, acc_sh, sid)
    pltpu.sync_copy(dstL_hbm.at[wid], dst_v)
    plsc.subcore_barrier()
    _deg_pass(dst_v, buf_a, acc_sh, sems)
    pltpu.sync_copy(dstN_hbm.at[wid], dst_v)
    _deg_pass(dst_v, buf_b, acc_sh, sems)
    plsc.subcore_barrier()
    _writeback(acc_sh, deg_out, cid, sid)
    plsc.subcore_barrier()

    for src_hbm, dst_hbm, agg_out in ((srcL_hbm, dstL_hbm, aggL_out),
                                      (srcN_hbm, dstN_hbm, aggN_out)):
        pltpu.sync_copy(src_hbm.at[wid], src_v)
        pltpu.sync_copy(dst_hbm.at[wid], dst_v)
        _zero_acc(z128_hbm, acc_sh, sid)
        plsc.subcore_barrier()
        _agg_pass(x_hbm, src_v, dst_v, bufs, acc_sh, sems)
        plsc.subcore_barrier()
        _writeback(acc_sh, agg_out, cid, sid)
        plsc.subcore_barrier()


_l1_sc = pl.kernel(
    _l1_body,
    out_type=(jax.ShapeDtypeStruct((NC, N_PAD, D), jnp.float32),
              jax.ShapeDtypeStruct((NC, N_PAD, D), jnp.float32),
              jax.ShapeDtypeStruct((NC, N_PAD, D), jnp.float32)),
    mesh=_mesh,
    scratch_types=[
        pltpu.VMEM((CHUNKS, CH), jnp.int32),
        pltpu.VMEM((CHUNKS, CH), jnp.int32),
        pltpu.VMEM((QCH, D), jnp.float32),
        pltpu.VMEM((QCH, D), jnp.float32),
        pltpu.VMEM((QCH, D), jnp.float32),
        pltpu.VMEM((QCH, D), jnp.float32),
        pltpu.VMEM_SHARED((N_PAD, D), jnp.float32),
        pltpu.SemaphoreType.DMA,
        pltpu.SemaphoreType.DMA,
        pltpu.SemaphoreType.DMA,
        pltpu.SemaphoreType.DMA,
    ],
)


def _agg_body(x_hbm, srcL_hbm, dstL_hbm, srcN_hbm, dstN_hbm, z128_hbm,
              aggL_out, aggN_out,
              src_v, dst_v, buf_a, buf_b, buf_c, buf_d, agg_sh,
              sem_a, sem_b, sem_c, sem_d):
    cid = lax.axis_index("c")
    sid = lax.axis_index("s")
    wid = sid * NC + cid
    sems = (sem_a, sem_b, sem_c, sem_d)
    bufs = (buf_a, buf_b, buf_c, buf_d)

    for src_hbm, dst_hbm, agg_out in ((srcL_hbm, dstL_hbm, aggL_out),
                                      (srcN_hbm, dstN_hbm, aggN_out)):
        pltpu.sync_copy(src_hbm.at[wid], src_v)
        pltpu.sync_copy(dst_hbm.at[wid], dst_v)
        _zero_acc(z128_hbm, agg_sh, sid)
        plsc.subcore_barrier()
        _agg_pass(x_hbm, src_v, dst_v, bufs, agg_sh, sems)
        plsc.subcore_barrier()
        _writeback(agg_sh, agg_out, cid, sid)
        plsc.subcore_barrier()


_agg_sc = pl.kernel(
    _agg_body,
    out_type=(jax.ShapeDtypeStruct((NC, N_PAD, D), jnp.float32),
              jax.ShapeDtypeStruct((NC, N_PAD, D), jnp.float32)),
    mesh=_mesh,
    scratch_types=[
        pltpu.VMEM((CHUNKS, CH), jnp.int32),
        pltpu.VMEM((CHUNKS, CH), jnp.int32),
        pltpu.VMEM((QCH, D), jnp.float32),
        pltpu.VMEM((QCH, D), jnp.float32),
        pltpu.VMEM((QCH, D), jnp.float32),
        pltpu.VMEM((QCH, D), jnp.float32),
        pltpu.VMEM_SHARED((N_PAD, D), jnp.float32),
        pltpu.SemaphoreType.DMA,
        pltpu.SemaphoreType.DMA,
        pltpu.SemaphoreType.DMA,
        pltpu.SemaphoreType.DMA,
    ],
)


def _combine_kernel(aggL_ref, aggN_ref, deg_ref, x_ref, w_ref,
                    b_ref, out_ref):
    aL = aggL_ref[0] + aggL_ref[1]
    aN = aggN_ref[0] + aggN_ref[1]
    dL = deg_ref[0, :, 0:1] + deg_ref[1, :, 0:1]
    dN = deg_ref[0, :, 64:65] + deg_ref[1, :, 64:65]
    mL = aL / jnp.maximum(dL, 1.0)
    mN = aN / jnp.maximum(dN, 1.0)
    acc = jnp.dot(mL, w_ref[0:D, :], preferred_element_type=jnp.float32)
    acc += jnp.dot(mN, w_ref[D:2 * D, :], preferred_element_type=jnp.float32)
    acc += jnp.dot(x_ref[...], w_ref[2 * D:3 * D, :],
                   preferred_element_type=jnp.float32)
    out_ref[...] = acc + b_ref[...]


_BLK = 1000


def _combine(aggL, aggN, deg, x, w, b):
    grid = (N // _BLK,)
    return pl.pallas_call(
        _combine_kernel,
        grid=grid,
        in_specs=[
            pl.BlockSpec((NC, _BLK, D), lambda i: (0, i, 0)),
            pl.BlockSpec((NC, _BLK, D), lambda i: (0, i, 0)),
            pl.BlockSpec((NC, _BLK, D), lambda i: (0, i, 0)),
            pl.BlockSpec((_BLK, D), lambda i: (i, 0)),
            pl.BlockSpec((3 * D, D), lambda i: (0, 0)),
            pl.BlockSpec((1, D), lambda i: (0, 0)),
        ],
        out_specs=pl.BlockSpec((_BLK, D), lambda i: (i, 0)),
        out_shape=jax.ShapeDtypeStruct((N, D), jnp.float32),
    )(aggL, aggN, deg, x, w, b)


def _prep_edges(edge_index):
    src = edge_index[0].astype(jnp.int32)
    dst = edge_index[1].astype(jnp.int32)
    pad = E_PAD - E
    ar = jnp.arange(pad, dtype=jnp.int32)
    pad_src = ar % N
    pad_dst = N + ar % (N_PAD - N)
    src_p = jnp.concatenate([src, pad_src]).reshape(NW, CHUNKS, CH)
    dst_p = jnp.concatenate([dst, pad_dst]).reshape(NW, CHUNKS, CH)
    return src_p, dst_p


def kernel(x, adj_low, adj_high, adj_nd_low, adj_nd_high,
           Wl1, Wr1, b1, Wlh1, Wrh1, bh1,
           Wl2, Wr2, b2, Wlh2, Wrh2, bh2):
    srcL, dstL = _prep_edges(adj_low)
    srcN, dstN = _prep_edges(adj_nd_low)
    z128 = jnp.zeros((N_PAD, D), jnp.float32)
    lane = jnp.arange(D, dtype=jnp.int32)
    ones2 = jnp.stack([jnp.where(lane < 64, 1.0, 0.0),
                       jnp.where(lane < 64, 0.0, 1.0)]).astype(jnp.float32)
    ones2 = jnp.broadcast_to(ones2[:, None, :], (2, QCH, D))

    w1 = jnp.concatenate([Wl1, ND_LAMBDA * Wlh1, Wr1 + ND_LAMBDA * Wrh1], axis=0)
    c1 = (b1 + ND_LAMBDA * bh1).reshape(1, D)
    w2 = jnp.concatenate([Wl2, ND_LAMBDA * Wlh2, Wr2 + ND_LAMBDA * Wrh2], axis=0)
    c2 = (b2 + ND_LAMBDA * bh2).reshape(1, D)

    deg, aggL1, aggN1 = _l1_sc(x, srcL, dstL, srcN, dstN, z128, ones2)
    h1 = _combine(aggL1, aggN1, deg, x, w1, c1)
    aggL2, aggN2 = _agg_sc(h1, srcL, dstL, srcN, dstN, z128)
    return _combine(aggL2, aggN2, deg, h1, w2, c2)


# speedup vs baseline: 8.4659x; 1.0019x over previous
import jax
import jax.numpy as jnp
from jax import lax
from jax.experimental import pallas as pl
from jax.experimental.pallas import tpu as pltpu
from jax.experimental.pallas import tpu_sc as plsc

N = 10000
E = 320000
D = 128
ND_LAMBDA = 0.5

NC = 2
NS = 16
NW = NC * NS
CH = 128
HCH = 64
EPT = ((E // NW + CH - 1) // CH) * CH
CHUNKS = EPT // CH
E_PAD = EPT * NW
N_PAD = 10112
R = N_PAD // NS

_mesh = plsc.VectorSubcoreMesh(core_axis_name="c", subcore_axis_name="s")


def _zero_acc(z128_hbm, acc_sh, sid):
    pltpu.sync_copy(z128_hbm.at[pl.ds(sid * R, R)],
                    acc_sh.at[pl.ds(sid * R, R)])


def _writeback(acc_sh, out, cid, sid):
    pltpu.sync_copy(acc_sh.at[pl.ds(sid * R, R)],
                    out.at[cid, pl.ds(sid * R, R)])


QCH = 16
NQ = CH // QCH


def _deg_pass(dst_v, ones_v, deg_sh, sems):
    def issue(j, q, sem):
        pltpu.async_copy(ones_v, deg_sh.at[dst_v.at[j, pl.ds(q * QCH, QCH)]],
                         sem, add=True)

    def wait(j, q, sem):
        pltpu.make_async_copy(
            ones_v, deg_sh.at[dst_v.at[j, pl.ds(q * QCH, QCH)]], sem).wait()

    for q in range(NQ):
        issue(0, q, sems[q])

    def step(j, carry):
        for q in range(NQ):
            wait(j, q, sems[q])
            issue(j + 1, q, sems[q])
        return carry

    lax.fori_loop(0, CHUNKS - 1, step, 0)
    for q in range(NQ):
        wait(CHUNKS - 1, q, sems[q])


def _agg_pass(x_hbm, src_v, dst_v, bufs, agg_sh, sems):
    def start(j, q, sem):
        pltpu.async_copy(x_hbm.at[src_v.at[j, pl.ds(q * QCH, QCH)]],
                         bufs[q], sem)

    def finish(j, q, sem):
        pltpu.make_async_copy(x_hbm.at[src_v.at[j, pl.ds(q * QCH, QCH)]],
                              bufs[q], sem).wait()
        pltpu.sync_copy(bufs[q],
                        agg_sh.at[dst_v.at[j, pl.ds(q * QCH, QCH)]],
                        add=True)

    for q in range(NQ):
        start(0, q, sems[q])

    def step(j, carry):
        for q in range(NQ):
            finish(j, q, sems[q])
            start(j + 1, q, sems[q])
        return carry

    lax.fori_loop(0, CHUNKS - 1, step, 0)
    for q in range(NQ):
        finish(CHUNKS - 1, q, sems[q])


def _l1_body(x_hbm, srcL_hbm, dstL_hbm, srcN_hbm, dstN_hbm, z128_hbm,
             ones2_hbm, deg_out, aggL_out, aggN_out,
             src_v, dst_v, buf_a, buf_b, buf_c, buf_d, buf_e, buf_f,
             buf_g, buf_h, acc_sh,
             sem_a, sem_b, sem_c, sem_d, sem_e, sem_f, sem_g, sem_h):
    cid = lax.axis_index("c")
    sid = lax.axis_index("s")
    wid = sid * NC + cid
    pltpu.sync_copy(ones2_hbm.at[0], buf_a)
    pltpu.sync_copy(ones2_hbm.at[1], buf_b)

    sems = (sem_a, sem_b, sem_c, sem_d, sem_e, sem_f, sem_g, sem_h)
    bufs = (buf_a, buf_b, buf_c, buf_d, buf_e, buf_f, buf_g, buf_h)
    _zero_acc(z128_hbm, acc_sh, sid)
    pltpu.sync_copy(dstL_hbm.at[wid], dst_v)
    plsc.subcore_barrier()
    _deg_pass(dst_v, buf_a, acc_sh, sems)
    pltpu.sync_copy(dstN_hbm.at[wid], dst_v)
    _deg_pass(dst_v, buf_b, acc_sh, sems)
    plsc.subcore_barrier()
    _writeback(acc_sh, deg_out, cid, sid)
    plsc.subcore_barrier()

    for src_hbm, dst_hbm, agg_out in ((srcL_hbm, dstL_hbm, aggL_out),
                                      (srcN_hbm, dstN_hbm, aggN_out)):
        pltpu.sync_copy(src_hbm.at[wid], src_v)
        pltpu.sync_copy(dst_hbm.at[wid], dst_v)
        _zero_acc(z128_hbm, acc_sh, sid)
        plsc.subcore_barrier()
        _agg_pass(x_hbm, src_v, dst_v, bufs, acc_sh, sems)
        plsc.subcore_barrier()
        _writeback(acc_sh, agg_out, cid, sid)
        plsc.subcore_barrier()


_l1_sc = pl.kernel(
    _l1_body,
    out_type=(jax.ShapeDtypeStruct((NC, N_PAD, D), jnp.float32),
              jax.ShapeDtypeStruct((NC, N_PAD, D), jnp.float32),
              jax.ShapeDtypeStruct((NC, N_PAD, D), jnp.float32)),
    mesh=_mesh,
    scratch_types=[
        pltpu.VMEM((CHUNKS, CH), jnp.int32),
        pltpu.VMEM((CHUNKS, CH), jnp.int32),
        pltpu.VMEM((QCH, D), jnp.float32),
        pltpu.VMEM((QCH, D), jnp.float32),
        pltpu.VMEM((QCH, D), jnp.float32),
        pltpu.VMEM((QCH, D), jnp.float32),
        pltpu.VMEM((QCH, D), jnp.float32),
        pltpu.VMEM((QCH, D), jnp.float32),
        pltpu.VMEM((QCH, D), jnp.float32),
        pltpu.VMEM((QCH, D), jnp.float32),
        pltpu.VMEM_SHARED((N_PAD, D), jnp.float32),
        pltpu.SemaphoreType.DMA,
        pltpu.SemaphoreType.DMA,
        pltpu.SemaphoreType.DMA,
        pltpu.SemaphoreType.DMA,
        pltpu.SemaphoreType.DMA,
        pltpu.SemaphoreType.DMA,
        pltpu.SemaphoreType.DMA,
        pltpu.SemaphoreType.DMA,
    ],
)


def _agg_body(x_hbm, srcL_hbm, dstL_hbm, srcN_hbm, dstN_hbm, z128_hbm,
              aggL_out, aggN_out,
              src_v, dst_v, buf_a, buf_b, buf_c, buf_d, buf_e, buf_f,
              buf_g, buf_h, agg_sh,
              sem_a, sem_b, sem_c, sem_d, sem_e, sem_f, sem_g, sem_h):
    cid = lax.axis_index("c")
    sid = lax.axis_index("s")
    wid = sid * NC + cid
    sems = (sem_a, sem_b, sem_c, sem_d, sem_e, sem_f, sem_g, sem_h)
    bufs = (buf_a, buf_b, buf_c, buf_d, buf_e, buf_f, buf_g, buf_h)

    for src_hbm, dst_hbm, agg_out in ((srcL_hbm, dstL_hbm, aggL_out),
                                      (srcN_hbm, dstN_hbm, aggN_out)):
        pltpu.sync_copy(src_hbm.at[wid], src_v)
        pltpu.sync_copy(dst_hbm.at[wid], dst_v)
        _zero_acc(z128_hbm, agg_sh, sid)
        plsc.subcore_barrier()
        _agg_pass(x_hbm, src_v, dst_v, bufs, agg_sh, sems)
        plsc.subcore_barrier()
        _writeback(agg_sh, agg_out, cid, sid)
        plsc.subcore_barrier()


_agg_sc = pl.kernel(
    _agg_body,
    out_type=(jax.ShapeDtypeStruct((NC, N_PAD, D), jnp.float32),
              jax.ShapeDtypeStruct((NC, N_PAD, D), jnp.float32)),
    mesh=_mesh,
    scratch_types=[
        pltpu.VMEM((CHUNKS, CH), jnp.int32),
        pltpu.VMEM((CHUNKS, CH), jnp.int32),
        pltpu.VMEM((QCH, D), jnp.float32),
        pltpu.VMEM((QCH, D), jnp.float32),
        pltpu.VMEM((QCH, D), jnp.float32),
        pltpu.VMEM((QCH, D), jnp.float32),
        pltpu.VMEM((QCH, D), jnp.float32),
        pltpu.VMEM((QCH, D), jnp.float32),
        pltpu.VMEM((QCH, D), jnp.float32),
        pltpu.VMEM((QCH, D), jnp.float32),
        pltpu.VMEM_SHARED((N_PAD, D), jnp.float32),
        pltpu.SemaphoreType.DMA,
        pltpu.SemaphoreType.DMA,
        pltpu.SemaphoreType.DMA,
        pltpu.SemaphoreType.DMA,
        pltpu.SemaphoreType.DMA,
        pltpu.SemaphoreType.DMA,
        pltpu.SemaphoreType.DMA,
        pltpu.SemaphoreType.DMA,
    ],
)


def _combine_kernel(aggL_ref, aggN_ref, deg_ref, x_ref, w_ref,
                    b_ref, out_ref):
    aL = aggL_ref[0] + aggL_ref[1]
    aN = aggN_ref[0] + aggN_ref[1]
    dL = deg_ref[0, :, 0:1] + deg_ref[1, :, 0:1]
    dN = deg_ref[0, :, 64:65] + deg_ref[1, :, 64:65]
    mL = aL / jnp.maximum(dL, 1.0)
    mN = aN / jnp.maximum(dN, 1.0)
    acc = jnp.dot(mL, w_ref[0:D, :], preferred_element_type=jnp.float32)
    acc += jnp.dot(mN, w_ref[D:2 * D, :], preferred_element_type=jnp.float32)
    acc += jnp.dot(x_ref[...], w_ref[2 * D:3 * D, :],
                   preferred_element_type=jnp.float32)
    out_ref[...] = acc + b_ref[...]


_BLK = 1000


def _combine(aggL, aggN, deg, x, w, b):
    grid = (N // _BLK,)
    return pl.pallas_call(
        _combine_kernel,
        grid=grid,
        in_specs=[
            pl.BlockSpec((NC, _BLK, D), lambda i: (0, i, 0)),
            pl.BlockSpec((NC, _BLK, D), lambda i: (0, i, 0)),
            pl.BlockSpec((NC, _BLK, D), lambda i: (0, i, 0)),
            pl.BlockSpec((_BLK, D), lambda i: (i, 0)),
            pl.BlockSpec((3 * D, D), lambda i: (0, 0)),
            pl.BlockSpec((1, D), lambda i: (0, 0)),
        ],
        out_specs=pl.BlockSpec((_BLK, D), lambda i: (i, 0)),
        out_shape=jax.ShapeDtypeStruct((N, D), jnp.float32),
    )(aggL, aggN, deg, x, w, b)


def _prep_edges(edge_index):
    src = edge_index[0].astype(jnp.int32)
    dst = edge_index[1].astype(jnp.int32)
    pad = E_PAD - E
    ar = jnp.arange(pad, dtype=jnp.int32)
    pad_src = ar % N
    pad_dst = N + ar % (N_PAD - N)
    src_p = jnp.concatenate([src, pad_src]).reshape(NW, CHUNKS, CH)
    dst_p = jnp.concatenate([dst, pad_dst]).reshape(NW, CHUNKS, CH)
    return src_p, dst_p


def kernel(x, adj_low, adj_high, adj_nd_low, adj_nd_high,
           Wl1, Wr1, b1, Wlh1, Wrh1, bh1,
           Wl2, Wr2, b2, Wlh2, Wrh2, bh2):
    srcL, dstL = _prep_edges(adj_low)
    srcN, dstN = _prep_edges(adj_nd_low)
    z128 = jnp.zeros((N_PAD, D), jnp.float32)
    lane = jnp.arange(D, dtype=jnp.int32)
    ones2 = jnp.stack([jnp.where(lane < 64, 1.0, 0.0),
                       jnp.where(lane < 64, 0.0, 1.0)]).astype(jnp.float32)
    ones2 = jnp.broadcast_to(ones2[:, None, :], (2, QCH, D))

    w1 = jnp.concatenate([Wl1, ND_LAMBDA * Wlh1, Wr1 + ND_LAMBDA * Wrh1], axis=0)
    c1 = (b1 + ND_LAMBDA * bh1).reshape(1, D)
    w2 = jnp.concatenate([Wl2, ND_LAMBDA * Wlh2, Wr2 + ND_LAMBDA * Wrh2], axis=0)
    c2 = (b2 + ND_LAMBDA * bh2).reshape(1, D)

    deg, aggL1, aggN1 = _l1_sc(x, srcL, dstL, srcN, dstN, z128, ones2)
    h1 = _combine(aggL1, aggN1, deg, x, w1, c1)
    aggL2, aggN2 = _agg_sc(h1, srcL, dstL, srcN, dstN, z128)
    return _combine(aggL2, aggN2, deg, h1, w2, c2)
